# baseline (speedup 1.0000x reference)
import jax
import jax.numpy as jnp
from jax.experimental import pallas as pl


def _mm_bf16(a, b):
    return jax.lax.dot_general(a, b, (((1,), (0,)), ((), ())),
                               preferred_element_type=jnp.float32)


def _matmul_ta_bf16(a, b):
    return jax.lax.dot_general(a, b, (((0,), (0,)), ((), ())),
                               preferred_element_type=jnp.float32)


def _split(v):
    hi = v.astype(jnp.bfloat16)
    lo = (v - hi.astype(jnp.float32)).astype(jnp.bfloat16)
    return hi, lo


def _mm3(hs, w):
    h1, h2 = hs
    w1, w2 = w
    return _mm_bf16(h1, w1) + (_mm_bf16(h1, w2) + _mm_bf16(h2, w1))


def _body(x_ref, A_ref, W1_ref, b1_ref, W2_ref, b2_ref, W3_ref, b3_ref,
          Wg1_ref, bg1_ref, Wg2_ref, bg2_ref, Wg3_ref, bg3_ref,
          Wg4_ref, bg4_ref, Wg5_ref, bg5_ref, Wg6_ref, bg6_ref, out_ref):
    n = A_ref.shape[0]
    M = (A_ref[...] != 0).astype(jnp.bfloat16)
    ones = jnp.ones((n, 1), jnp.bfloat16)
    deg = _matmul_ta_bf16(M, ones) + 1.0
    dinv = jax.lax.rsqrt(deg)
    dinv2 = dinv * dinv

    W1 = _split(W1_ref[...])
    W2 = _split(W2_ref[...])
    W3 = _split(W3_ref[...])
    Wg1 = _split(Wg1_ref[...])
    Wg2 = _split(Wg2_ref[...])
    Wg3 = _split(Wg3_ref[...])
    Wg4 = _split(Wg4_ref[...])
    Wg5 = _split(Wg5_ref[...])
    Wg6 = _split(Wg6_ref[...])
    b1, b2, b3 = b1_ref[...], b2_ref[...], b3_ref[...]

    W1g = _split(_mm3(_split(W1_ref[...]), Wg1))
    W2g = _split(_mm3(_split(W2_ref[...]), Wg2))
    W3g = _split(_mm3(_split(W3_ref[...]), Wg3))
    b1g = _mm3(_split(b1), Wg1)
    b2g = _mm3(_split(b2), Wg2)
    b3g = _mm3(_split(b3), Wg3)

    def nprop(hw, bb):
        t = _matmul_ta_bf16(M, (hw * dinv).astype(jnp.bfloat16))
        return t * dinv + hw * dinv2 + bb

    xs = _split(x_ref[...])
    x1l = _mm3(xs, W1) + b1
    hw1 = _mm3(xs, W1g) + b1g
    x1 = x1l + jax.nn.relu(nprop(hw1, bg1_ref[...]))

    x1s = _split(x1)
    x2l = _mm3(x1s, W2) + b2
    hw2 = _mm3(x1s, W2g) + b2g
    x2 = x2l + jax.nn.relu(nprop(hw2, bg2_ref[...]))

    x2s = _split(x2)
    x3l = _mm3(x2s, W3) + b3
    hw3 = _mm3(x2s, W3g) + b3g
    x3 = x3l + 0.5 * jax.nn.relu(nprop(hw3, bg3_ref[...]))

    x3s = _split(x3)
    x4 = x3 + 0.5 * jax.nn.relu(nprop(_mm3(x3s, Wg4), bg4_ref[...]))
    x4s = _split(x4)
    x5 = x4 + 0.25 * jax.nn.relu(nprop(_mm3(x4s, Wg5), bg5_ref[...]))
    x5s = _split(x5)
    out_ref[...] = x5 + 0.25 * nprop(_mm3(x5s, Wg6), bg6_ref[...])


def kernel(x, A_neg, A_pos, W1, b1, W2, b2, W3, b3, Wg1, bg1, Wg2, bg2,
           Wg3, bg3, Wg4, bg4, Wg5, bg5, Wg6, bg6):
    del A_pos
    n, dout = x.shape[0], Wg3.shape[0]
    row = lambda v: v.reshape(1, -1)
    return pl.pallas_call(
        _body,
        out_shape=jax.ShapeDtypeStruct((n, dout), jnp.float32),
    )(x, A_neg, W1, row(b1), W2, row(b2), W3, row(b3),
      Wg1, row(bg1), Wg2, row(bg2), Wg3, row(bg3),
      Wg4, row(bg4), Wg5, row(bg5), Wg6, row(bg6))
